# Initial kernel scaffold; baseline (speedup 1.0000x reference)
import jax
import jax.numpy as jnp
from jax.experimental import pallas as pl


def kernel(x, Wo, bo, Wm, bm, weight, Wr1, br1, Wr2, br2):
    raise NotImplementedError("write your pallas kernel here")


# speedup vs baseline: 4539.6393x; 4539.6393x over previous
import functools

import jax
import jax.numpy as jnp
from jax import lax
from jax.experimental import pallas as pl
from jax.experimental.pallas import tpu as pltpu
from jax.experimental.pallas import tpu_sc as plsc

_NC, _NS, _L = 2, 16, 16
_NW = _NC * _NS


def _stage1_body(x_ref, w_ref, b_ref, sidx_ref, sval_ref, *, T, K):
    xb = x_ref[0]
    C = xb.shape[0]
    zcol = jnp.zeros((C, 1), jnp.float32)
    xm1 = jnp.concatenate([zcol, xb[:, : T - 1]], axis=1)
    xp1 = jnp.concatenate([xb[:, 1:], zcol], axis=1)
    y = (
        jnp.dot(w_ref[0], xm1, preferred_element_type=jnp.float32)
        + jnp.dot(w_ref[1], xb, preferred_element_type=jnp.float32)
        + jnp.dot(w_ref[2], xp1, preferred_element_type=jnp.float32)
        + b_ref[...]
    )
    t_iota = lax.broadcasted_iota(jnp.int32, (16, T), 1).astype(jnp.float32)
    k_row = lax.broadcasted_iota(jnp.int32, (16, T), 0).astype(
        jnp.float32
    ) - float(K // 2)
    pos = jnp.clip(t_iota + k_row + y, 0.0, float(T - 1))
    pf = jnp.floor(pos)
    pc = jnp.ceil(pos)
    wf = pc - pos
    wc = pos - pf
    mod = jax.nn.sigmoid(y)
    mod_sh = jnp.concatenate([mod[8:16], mod[0:8]], axis=0)
    af = wf * mod_sh
    ac = wc * mod_sh
    gbase = pl.program_id(0) * T
    fi = pf.astype(jnp.int32) + gbase
    ci = pc.astype(jnp.int32) + gbase
    sidx_ref[0] = jnp.concatenate([fi[0:8], ci[0:8]], axis=0)
    sval_ref[0] = jnp.concatenate([af[0:8], ac[0:8]], axis=0)


def _stage1(x, w16t, b16):
    B, C, T = x.shape
    K = 5
    return pl.pallas_call(
        functools.partial(_stage1_body, T=T, K=K),
        grid=(B,),
        in_specs=[
            pl.BlockSpec((1, C, T), lambda b: (b, 0, 0)),
            pl.BlockSpec((3, 16, C), lambda b: (0, 0, 0)),
            pl.BlockSpec((16, 1), lambda b: (0, 0)),
        ],
        out_specs=[
            pl.BlockSpec((1, 16, T), lambda b: (b, 0, 0)),
            pl.BlockSpec((1, 16, T), lambda b: (b, 0, 0)),
        ],
        out_shape=[
            jax.ShapeDtypeStruct((B, 16, T), jnp.int32),
            jax.ShapeDtypeStruct((B, 16, T), jnp.float32),
        ],
    )(x, w16t, b16)


def _sc_gather(xt, sidx, sval, wd16):
    ROWS, C = xt.shape
    J = 10
    K = J // 2
    RPW = ROWS // _NW
    CH = 8
    NCH = RPW // CH
    NCG = C // _L
    mesh = plsc.VectorSubcoreMesh(
        core_axis_name="c", subcore_axis_name="s",
        num_cores=_NC, num_subcores=_NS,
    )

    def body(xt_hbm, sidx_hbm, sval_hbm, wd_hbm, out_hbm,
             idx_v, sval_v, g_v, out_v, wd_v, sem):
        wid = lax.axis_index("s") * _NC + lax.axis_index("c")
        base = wid * RPW
        pltpu.sync_copy(wd_hbm, wd_v)
        pltpu.sync_copy(sidx_hbm.at[pl.ds(base * J, RPW * J)], idx_v)
        pltpu.sync_copy(sval_hbm.at[pl.ds(base * J, RPW * J)], sval_v)

        def chunk(i, carry):
            rb = i * CH
            pltpu.async_copy(
                xt_hbm.at[idx_v.at[pl.ds(rb * J, CH * J)]], g_v, sem
            ).wait()

            def row(r, carry2):
                sp = [
                    plsc.load_gather(
                        sval_v,
                        [lax.broadcast((rb + r) * J + j, (16,))],
                    )
                    for j in range(J)
                ]

                def col(cg, carry3):
                    co = cg * _L
                    acc = jnp.zeros((_L,), jnp.float32)
                    for j in range(J):
                        g = g_v[r * J + j, pl.ds(co, _L)]
                        w = wd_v[j, pl.ds(co, _L)]
                        acc = acc + sp[j] * w * g
                    out_v[r, pl.ds(co, _L)] = acc
                    return carry3

                return lax.fori_loop(0, NCG, col, carry2)

            lax.fori_loop(0, CH, row, 0)
            pltpu.sync_copy(out_v, out_hbm.at[pl.ds(base + rb, CH)])
            return carry

        lax.fori_loop(0, NCH, chunk, 0)

    run = pl.kernel(
        body,
        out_type=jax.ShapeDtypeStruct((ROWS, C), jnp.float32),
        mesh=mesh,
        scratch_types=[
            pltpu.VMEM((RPW * J,), jnp.int32),
            pltpu.VMEM((RPW * J,), jnp.float32),
            pltpu.VMEM((CH * J, C), jnp.float32),
            pltpu.VMEM((CH, C), jnp.float32),
            pltpu.VMEM((16, C), jnp.float32),
            pltpu.SemaphoreType.DMA,
        ],
        compiler_params=pltpu.CompilerParams(needs_layout_passes=False),
    )
    return run(xt, sidx, sval, wd16)


def _stage3_body(d_ref, w_ref, b1_ref, w2_ref, b2_ref, rec_ref):
    db = d_ref[0]
    T = db.shape[0]
    h0 = jnp.dot(db, w_ref[0], preferred_element_type=jnp.float32)
    h1 = jnp.dot(db, w_ref[1], preferred_element_type=jnp.float32)
    h2 = jnp.dot(db, w_ref[2], preferred_element_type=jnp.float32)
    zrow = jnp.zeros((1, h0.shape[1]), jnp.float32)
    y = (
        h1
        + jnp.concatenate([zrow, h0[: T - 1]], axis=0)
        + jnp.concatenate([h2[1:], zrow], axis=0)
        + b1_ref[...]
    )
    h = jnp.maximum(y, 0.0)
    rec = jnp.dot(h, w2_ref[...], preferred_element_type=jnp.float32) + b2_ref[...]
    rec_ref[0] = rec


def _stage3(dt3, wr1t, br1, w2, b2):
    B, T, C = dt3.shape
    C2 = wr1t.shape[-1]
    return pl.pallas_call(
        _stage3_body,
        grid=(B,),
        in_specs=[
            pl.BlockSpec((1, T, C), lambda b: (b, 0, 0)),
            pl.BlockSpec((3, C, C2), lambda b: (0, 0, 0)),
            pl.BlockSpec((1, C2), lambda b: (0, 0)),
            pl.BlockSpec((C2, 1), lambda b: (0, 0)),
            pl.BlockSpec((1, 1), lambda b: (0, 0)),
        ],
        out_specs=pl.BlockSpec((1, T, 1), lambda b: (b, 0, 0)),
        out_shape=jax.ShapeDtypeStruct((B, T, 1), jnp.float32),
    )(dt3, wr1t, br1, w2, b2)


def kernel(x, Wo, bo, Wm, bm, weight, Wr1, br1, Wr2, br2):
    B, C, T = x.shape
    K = weight.shape[-1]
    w16 = (
        jnp.zeros((16, C, 3), jnp.float32)
        .at[0:K].set(Wo[0:K])
        .at[8 : 8 + K].set(Wm)
    )
    b16 = (
        jnp.zeros((16,), jnp.float32).at[0:K].set(bo[0:K]).at[8 : 8 + K].set(bm)
    )
    w16t = jnp.transpose(w16, (2, 0, 1))
    sidx16, sval16 = _stage1(x, w16t, b16[:, None])
    sidx = jnp.concatenate([sidx16[:, 0:K], sidx16[:, 8 : 8 + K]], axis=1)
    sval = jnp.concatenate([sval16[:, 0:K], sval16[:, 8 : 8 + K]], axis=1)
    sidx = jnp.transpose(sidx, (0, 2, 1)).reshape(B * T * 2 * K)
    sval = jnp.transpose(sval, (0, 2, 1)).reshape(B * T * 2 * K)
    xt = jnp.transpose(x, (0, 2, 1)).reshape(B * T, C)
    wdiag = weight[jnp.arange(C), jnp.arange(C), :]
    wd16 = (
        jnp.zeros((16, C), jnp.float32)
        .at[0:K].set(wdiag.T)
        .at[K : 2 * K].set(wdiag.T)
    )
    dt = _sc_gather(xt, sidx, sval, wd16)
    dt3 = dt.reshape(B, T, C)
    deformed = jnp.transpose(dt3, (0, 2, 1))
    wr1t = jnp.transpose(Wr1, (2, 1, 0))
    rec3 = _stage3(dt3, wr1t, br1[None, :], Wr2[0], br2[None, :])
    rec = rec3.reshape(B, T)
    return deformed, rec
